# baseline (speedup 1.0000x reference)
import functools

import jax
import jax.numpy as jnp
import numpy as np
from jax import lax
from jax.experimental import pallas as pl
from jax.experimental.pallas import tpu as pltpu, tpu_sc as plsc

D_MODEL = 256
N_FRAMES = 3
N_LEVELS = 4
N_POINTS = 4
MH = 64
DH = 4
SPATIAL = ((64, 64), (32, 32), (16, 16), (8, 8))
LSI = (0, 4096, 5120, 5376)
S_FRAME = 5440
LEN_IN = S_FRAME * N_FRAMES
LEN_PAD = 16384
N_B = 2
LQ = 2048
NM = N_B * MH

TILE_V = 1024
TQ = 128
QC = 512
N_WORKERS = 32


def _value_proj_body(x_ref, w_ref, b_ref, o_ref):
    v = jnp.dot(x_ref[0], w_ref[...], preferred_element_type=jnp.float32, precision=lax.Precision.HIGHEST) + b_ref[...]
    bf = v.T.astype(jnp.bfloat16)
    lo = lax.bitcast_convert_type(bf[:128], jnp.uint16).astype(jnp.int32)
    hi = lax.bitcast_convert_type(bf[128:], jnp.uint16).astype(jnp.int32)
    o_ref[...] = (lo | (hi << 16)).reshape(MH, DH // 2, TILE_V)


def _value_proj(x, W, b):
    grid = (N_B, LEN_PAD // TILE_V)
    return pl.pallas_call(
        _value_proj_body,
        grid=grid,
        in_specs=[
            pl.BlockSpec((1, TILE_V, D_MODEL), lambda n, i: (n, i, 0)),
            pl.BlockSpec((D_MODEL, D_MODEL), lambda n, i: (0, 0)),
            pl.BlockSpec((D_MODEL,), lambda n, i: (0,)),
        ],
        out_specs=pl.BlockSpec((MH, DH // 2, TILE_V), lambda n, i: (n, 0, i)),
        out_shape=jax.ShapeDtypeStruct((NM, DH // 2, LEN_PAD), jnp.int32),
    )(x, W, b)


def _payload_body(q_ref, rp_ref, wo_ref, bo_ref, wa_ref, ba_ref, sc_ref,
                  gs_ref, ge_ref, xyz_ref, att_ref):
    q = q_ref[0]
    off = jnp.dot(q, wo_ref[...], preferred_element_type=jnp.float32, precision=lax.Precision.HIGHEST) + bo_ref[...]
    refb = jnp.dot(rp_ref[0], sc_ref[...], preferred_element_type=jnp.float32, precision=lax.Precision.HIGHEST)
    xyz = refb + off
    logits = jnp.dot(q, wa_ref[...], preferred_element_type=jnp.float32, precision=lax.Precision.HIGHEST) + ba_ref[...]
    e = jnp.exp(logits)
    s = jnp.dot(e, gs_ref[...], preferred_element_type=jnp.float32, precision=lax.Precision.HIGHEST)
    sm = e * jnp.dot(1.0 / s, ge_ref[...], preferred_element_type=jnp.float32, precision=lax.Precision.HIGHEST)
    xyz_ref[...] = xyz.T.reshape(1, MH, N_LEVELS * N_POINTS * 3, TQ)
    att_ref[...] = sm.T.reshape(1, MH, N_LEVELS * N_POINTS, TQ)


def _payload(query, rp12, W_off, b_off2, W_attn, b_attn, scale_mat, gsum, gexp):
    grid = (N_B, LQ // TQ)
    LP = N_LEVELS * N_POINTS
    return pl.pallas_call(
        _payload_body,
        grid=grid,
        in_specs=[
            pl.BlockSpec((1, TQ, D_MODEL), lambda n, i: (n, i, 0)),
            pl.BlockSpec((1, TQ, 12), lambda n, i: (n, i, 0)),
            pl.BlockSpec((D_MODEL, 3072), lambda n, i: (0, 0)),
            pl.BlockSpec((3072,), lambda n, i: (0,)),
            pl.BlockSpec((D_MODEL, 1024), lambda n, i: (0, 0)),
            pl.BlockSpec((1024,), lambda n, i: (0,)),
            pl.BlockSpec((12, 3072), lambda n, i: (0, 0)),
            pl.BlockSpec((1024, MH), lambda n, i: (0, 0)),
            pl.BlockSpec((MH, 1024), lambda n, i: (0, 0)),
        ],
        out_specs=[
            pl.BlockSpec((1, MH, LP * 3, TQ), lambda n, i: (n, 0, 0, i)),
            pl.BlockSpec((1, MH, LP, TQ), lambda n, i: (n, 0, 0, i)),
        ],
        out_shape=[
            jax.ShapeDtypeStruct((N_B, MH, LP * 3, LQ), jnp.float32),
            jax.ShapeDtypeStruct((N_B, MH, LP, LQ), jnp.float32),
        ],
    )(query, rp12, W_off, b_off2, W_attn, b_attn, scale_mat, gsum, gexp)


def _floor16(v):
    vi = v.astype(jnp.int32)
    vf = vi.astype(jnp.float32)
    neg = v < vf
    return jnp.where(neg, vi - 1, vi), jnp.where(neg, vf - 1.0, vf)


def _sc_sample(value_t, xyz, att):
    mesh = plsc.VectorSubcoreMesh(core_axis_name="c", subcore_axis_name="s")
    LP = N_LEVELS * N_POINTS
    n_pairs = NM // N_WORKERS

    @functools.partial(
        pl.kernel,
        out_type=jax.ShapeDtypeStruct((NM, DH, LQ), jnp.float32),
        mesh=mesh,
        compiler_params=pltpu.CompilerParams(needs_layout_passes=False),
        scratch_types=[
            pltpu.VMEM(((DH // 2) * LEN_PAD,), jnp.int32),
            pltpu.VMEM((LP * 3, QC), jnp.float32),
            pltpu.VMEM((LP, QC), jnp.float32),
            pltpu.VMEM((DH, LQ), jnp.float32),
        ],
    )
    def body(value_hbm, xyz_hbm, att_hbm, out_hbm, table, xyzv, attv, outv):
        wid = lax.axis_index("s") * 2 + lax.axis_index("c")

        def pair_body(p, _):
            nm = wid * n_pairs + p
            pltpu.sync_copy(value_hbm.at[nm], table)

            def qc_body(qc, _):
                qb = pl.multiple_of(qc * QC, QC)
                pltpu.sync_copy(xyz_hbm.at[nm, :, pl.ds(qb, QC)], xyzv)
                pltpu.sync_copy(att_hbm.at[nm, :, pl.ds(qb, QC)], attv)

                @plsc.parallel_loop(0, QC // 16, 1, unroll=2)
                def qv_body(qv):
                    for u in range(1):
                        qo = pl.multiple_of(qv * 16 + u * 16, 16)
                        acc = [jnp.zeros((16,), jnp.float32) for _ in range(DH)]
                        for lvl in range(N_LEVELS):
                            H, W = SPATIAL[lvl]
                            base = LSI[lvl]
                            for pt in range(N_POINTS):
                                lp = lvl * N_POINTS + pt
                                x = xyzv[lp * 3 + 0, pl.ds(qo, 16)]
                                y = xyzv[lp * 3 + 1, pl.ds(qo, 16)]
                                t = xyzv[lp * 3 + 2, pl.ds(qo, 16)]
                                a = attv[lp, pl.ds(qo, 16)]
                                x0, x0f = _floor16(x)
                                y0, y0f = _floor16(y)
                                t0, t0f = _floor16(t)
                                fx = x - x0f
                                fy = y - y0f
                                ft = t - t0f
                                wx0 = jnp.where((x0 >= 0) & (x0 < W), 1.0 - fx, 0.0)
                                wx1 = jnp.where((x0 >= -1) & (x0 < W - 1), fx, 0.0)
                                wy0 = jnp.where((y0 >= 0) & (y0 < H), 1.0 - fy, 0.0)
                                wy1 = jnp.where((y0 >= -1) & (y0 < H - 1), fy, 0.0)
                                wt0 = jnp.where((t0 >= 0) & (t0 < N_FRAMES), 1.0 - ft, 0.0) * a
                                wt1 = jnp.where((t0 >= -1) & (t0 < N_FRAMES - 1), ft, 0.0) * a
                                xc0 = jnp.clip(x0, 0, W - 1)
                                xc1 = jnp.clip(x0 + 1, 0, W - 1)
                                yc0 = jnp.clip(y0, 0, H - 1) * W
                                yc1 = jnp.clip(y0 + 1, 0, H - 1) * W
                                tc0 = jnp.clip(t0, 0, N_FRAMES - 1) * S_FRAME + base
                                tc1 = jnp.clip(t0 + 1, 0, N_FRAMES - 1) * S_FRAME + base
                                for (r, wr) in ((tc0 + yc0, wt0 * wy0),
                                                (tc0 + yc1, wt0 * wy1),
                                                (tc1 + yc0, wt1 * wy0),
                                                (tc1 + yc1, wt1 * wy1)):
                                    for (xc, wx) in ((xc0, wx0), (xc1, wx1)):
                                        idx = r + xc
                                        w = wr * wx
                                        for pp in range(DH // 2):
                                            gw = plsc.load_gather(table, [idx + (pp * LEN_PAD)])
                                            ve, vo = plsc.unpack(
                                                plsc.bitcast(gw, jnp.bfloat16),
                                                format=plsc.PackFormat.INTERLEAVED)
                                            acc[2 * pp] = acc[2 * pp] + ve * w
                                            acc[2 * pp + 1] = acc[2 * pp + 1] + vo * w
                        for dd in range(DH):
                            outv[dd, pl.ds(qb + qo, 16)] = acc[dd]

                return 0

            lax.fori_loop(0, LQ // QC, qc_body, 0)
            pltpu.sync_copy(outv, out_hbm.at[nm])
            return 0

        lax.fori_loop(0, n_pairs, pair_body, 0)

    return body(value_t, xyz, att)


def _out_proj_body(s_ref, w_ref, b_ref, o_ref):
    y = lax.dot_general(s_ref[0], w_ref[...], (((0,), (0,)), ((), ())),
                        preferred_element_type=jnp.float32, precision=lax.Precision.HIGHEST)
    o_ref[...] = (y + b_ref[...])[None]


def _out_proj(sc_out, W, b):
    return pl.pallas_call(
        _out_proj_body,
        grid=(N_B,),
        in_specs=[
            pl.BlockSpec((1, D_MODEL, LQ), lambda n: (n, 0, 0)),
            pl.BlockSpec((D_MODEL, D_MODEL), lambda n: (0, 0)),
            pl.BlockSpec((D_MODEL,), lambda n: (0,)),
        ],
        out_specs=pl.BlockSpec((1, LQ, D_MODEL), lambda n: (n, 0, 0)),
        out_shape=jax.ShapeDtypeStruct((N_B, LQ, D_MODEL), jnp.float32),
    )(sc_out, W, b)


def _consts():
    sc = np.zeros((12, 3072), np.float32)
    for m in range(MH):
        for lvl in range(N_LEVELS):
            H, W = SPATIAL[lvl]
            s3 = (W, H, N_FRAMES)
            for p in range(N_POINTS):
                for c in range(3):
                    sc[lvl * 3 + c, ((m * N_LEVELS + lvl) * N_POINTS + p) * 3 + c] = s3[c]
    gsum = np.zeros((1024, MH), np.float32)
    gexp = np.zeros((MH, 1024), np.float32)
    for i in range(1024):
        gsum[i, i // 16] = 1.0
        gexp[i // 16, i] = 1.0
    return jnp.asarray(sc), jnp.asarray(gsum), jnp.asarray(gexp)


def kernel(query, reference_points, input_flatten, input_spatial_shapes,
           input_level_start_index, W_value, b_value, W_offsets, b_offsets,
           W_attn, b_attn, W_out, b_out):
    scale_mat, gsum, gexp = _consts()
    perm = np.concatenate([np.arange(0, D_MODEL, 2), np.arange(1, D_MODEL, 2)])
    value_t = _value_proj(input_flatten, W_value[:, perm], b_value[perm])
    rp12 = reference_points.reshape(N_B, LQ, 12)
    xyz, att = _payload(query, rp12, W_offsets, b_offsets - 0.5, W_attn, b_attn,
                        scale_mat, gsum, gexp)
    sc_out = _sc_sample(value_t.reshape(NM, (DH // 2) * LEN_PAD),
                        xyz.reshape(NM, N_LEVELS * N_POINTS * 3, LQ),
                        att.reshape(NM, N_LEVELS * N_POINTS, LQ))
    return _out_proj(sc_out.reshape(N_B, D_MODEL, LQ), W_out, b_out)


# speedup vs baseline: 1.7565x; 1.7565x over previous
import functools

import jax
import jax.numpy as jnp
import numpy as np
from jax import lax
from jax.experimental import pallas as pl
from jax.experimental.pallas import tpu as pltpu, tpu_sc as plsc

D_MODEL = 256
N_FRAMES = 3
N_LEVELS = 4
N_POINTS = 4
MH = 64
DH = 4
SPATIAL = ((64, 64), (32, 32), (16, 16), (8, 8))
LSI = (0, 4096, 5120, 5376)
S_FRAME = 5440
LEN_IN = S_FRAME * N_FRAMES
LEN_PAD = 16384
N_B = 2
LQ = 2048
NM = N_B * MH

TILE_V = 1024
TQ = 128
QC = 512
N_WORKERS = 32


def _value_proj_body(x_ref, w_ref, b_ref, o_ref):
    v = jnp.dot(x_ref[0], w_ref[...], preferred_element_type=jnp.float32, precision=lax.Precision.HIGHEST) + b_ref[...]
    bf = v.T.astype(jnp.bfloat16)
    lo = lax.bitcast_convert_type(bf[:128], jnp.uint16).astype(jnp.int32)
    hi = lax.bitcast_convert_type(bf[128:], jnp.uint16).astype(jnp.int32)
    o_ref[...] = (lo | (hi << 16)).reshape(MH, DH // 2, TILE_V)


def _value_proj(x, W, b):
    grid = (N_B, LEN_PAD // TILE_V)
    return pl.pallas_call(
        _value_proj_body,
        grid=grid,
        in_specs=[
            pl.BlockSpec((1, TILE_V, D_MODEL), lambda n, i: (n, i, 0)),
            pl.BlockSpec((D_MODEL, D_MODEL), lambda n, i: (0, 0)),
            pl.BlockSpec((D_MODEL,), lambda n, i: (0,)),
        ],
        out_specs=pl.BlockSpec((MH, DH // 2, TILE_V), lambda n, i: (n, 0, i)),
        out_shape=jax.ShapeDtypeStruct((NM, DH // 2, LEN_PAD), jnp.int32),
    )(x, W, b)


def _payload_body(q_ref, rp_ref, wo_ref, bo_ref, wa_ref, ba_ref, sc_ref,
                  gs_ref, ge_ref, xyz_ref, att_ref):
    q = q_ref[0]
    off = jnp.dot(q, wo_ref[...], preferred_element_type=jnp.float32, precision=lax.Precision.HIGHEST) + bo_ref[...]
    refb = jnp.dot(rp_ref[0], sc_ref[...], preferred_element_type=jnp.float32, precision=lax.Precision.HIGHEST)
    xyz = refb + off
    logits = jnp.dot(q, wa_ref[...], preferred_element_type=jnp.float32, precision=lax.Precision.HIGHEST) + ba_ref[...]
    e = jnp.exp(logits)
    s = jnp.dot(e, gs_ref[...], preferred_element_type=jnp.float32, precision=lax.Precision.HIGHEST)
    sm = e * jnp.dot(1.0 / s, ge_ref[...], preferred_element_type=jnp.float32, precision=lax.Precision.HIGHEST)
    xyz_ref[...] = xyz.T.reshape(1, MH, N_LEVELS * N_POINTS * 3, TQ)
    att_ref[...] = sm.T.reshape(1, MH, N_LEVELS * N_POINTS, TQ)


def _payload(query, rp12, W_off, b_off2, W_attn, b_attn, scale_mat, gsum, gexp):
    grid = (N_B, LQ // TQ)
    LP = N_LEVELS * N_POINTS
    return pl.pallas_call(
        _payload_body,
        grid=grid,
        in_specs=[
            pl.BlockSpec((1, TQ, D_MODEL), lambda n, i: (n, i, 0)),
            pl.BlockSpec((1, TQ, 12), lambda n, i: (n, i, 0)),
            pl.BlockSpec((D_MODEL, 3072), lambda n, i: (0, 0)),
            pl.BlockSpec((3072,), lambda n, i: (0,)),
            pl.BlockSpec((D_MODEL, 1024), lambda n, i: (0, 0)),
            pl.BlockSpec((1024,), lambda n, i: (0,)),
            pl.BlockSpec((12, 3072), lambda n, i: (0, 0)),
            pl.BlockSpec((1024, MH), lambda n, i: (0, 0)),
            pl.BlockSpec((MH, 1024), lambda n, i: (0, 0)),
        ],
        out_specs=[
            pl.BlockSpec((1, MH, LP * 3, TQ), lambda n, i: (n, 0, 0, i)),
            pl.BlockSpec((1, MH, LP, TQ), lambda n, i: (n, 0, 0, i)),
        ],
        out_shape=[
            jax.ShapeDtypeStruct((N_B, MH, LP * 3, LQ), jnp.float32),
            jax.ShapeDtypeStruct((N_B, MH, LP, LQ), jnp.float32),
        ],
    )(query, rp12, W_off, b_off2, W_attn, b_attn, scale_mat, gsum, gexp)


def _floor16(v):
    vi = v.astype(jnp.int32)
    vf = vi.astype(jnp.float32)
    neg = v < vf
    return jnp.where(neg, vi - 1, vi), jnp.where(neg, vf - 1.0, vf)


def _sc_sample(value_t, xyz, att, lvlc):
    mesh = plsc.VectorSubcoreMesh(core_axis_name="c", subcore_axis_name="s")
    LP = N_LEVELS * N_POINTS
    n_pairs = NM // N_WORKERS

    @functools.partial(
        pl.kernel,
        out_type=jax.ShapeDtypeStruct((NM, DH, LQ), jnp.float32),
        mesh=mesh,
        compiler_params=pltpu.CompilerParams(needs_layout_passes=False),
        scratch_types=[
            pltpu.VMEM(((DH // 2) * LEN_PAD,), jnp.int32),
            pltpu.VMEM((LP * 3, QC), jnp.float32),
            pltpu.VMEM((LP, QC), jnp.float32),
            pltpu.VMEM((DH, LQ), jnp.float32),
            pltpu.VMEM((LP * 5, 16), jnp.int32),
        ],
    )
    def body(value_hbm, xyz_hbm, att_hbm, lvlc_hbm, out_hbm,
             table, xyzv, attv, outv, lvlv):
        wid = lax.axis_index("s") * 2 + lax.axis_index("c")
        pltpu.sync_copy(lvlc_hbm, lvlv)

        def lp_body(args, acc):
            lp, qo = args
            a0, a1, a2, a3 = acc
            Wi = lvlv[5 * lp + 0]
            Wm1 = lvlv[5 * lp + 1]
            Hm1 = lvlv[5 * lp + 2]
            basev = lvlv[5 * lp + 3]
            x = xyzv[3 * lp + 0, pl.ds(qo, 16)]
            y = xyzv[3 * lp + 1, pl.ds(qo, 16)]
            t = xyzv[3 * lp + 2, pl.ds(qo, 16)]
            a = attv[lp, pl.ds(qo, 16)]
            x0, x0f = _floor16(x)
            y0, y0f = _floor16(y)
            t0, t0f = _floor16(t)
            fx = x - x0f
            fy = y - y0f
            ft = t - t0f
            wx0 = jnp.where((x0 >= 0) & (x0 <= Wm1), 1.0 - fx, 0.0)
            wx1 = jnp.where((x0 >= -1) & (x0 < Wm1), fx, 0.0)
            wy0 = jnp.where((y0 >= 0) & (y0 <= Hm1), 1.0 - fy, 0.0)
            wy1 = jnp.where((y0 >= -1) & (y0 < Hm1), fy, 0.0)
            wt0 = jnp.where((t0 >= 0) & (t0 < N_FRAMES), 1.0 - ft, 0.0) * a
            wt1 = jnp.where((t0 >= -1) & (t0 < N_FRAMES - 1), ft, 0.0) * a
            zero = jnp.zeros((16,), jnp.int32)
            xc0 = jnp.minimum(jnp.maximum(x0, zero), Wm1)
            xc1 = jnp.minimum(jnp.maximum(x0 + 1, zero), Wm1)
            yc0 = jnp.minimum(jnp.maximum(y0, zero), Hm1) * Wi
            yc1 = jnp.minimum(jnp.maximum(y0 + 1, zero), Hm1) * Wi
            tc0 = jnp.clip(t0, 0, N_FRAMES - 1) * S_FRAME + basev
            tc1 = jnp.clip(t0 + 1, 0, N_FRAMES - 1) * S_FRAME + basev
            for (r, wr) in ((tc0 + yc0, wt0 * wy0),
                            (tc0 + yc1, wt0 * wy1),
                            (tc1 + yc0, wt1 * wy0),
                            (tc1 + yc1, wt1 * wy1)):
                for (xc, wx) in ((xc0, wx0), (xc1, wx1)):
                    idx = r + xc
                    w = wr * wx
                    gw0 = plsc.load_gather(table, [idx])
                    ve, vo = plsc.unpack(plsc.bitcast(gw0, jnp.bfloat16),
                                         format=plsc.PackFormat.INTERLEAVED)
                    a0 = a0 + ve * w
                    a1 = a1 + vo * w
                    gw1 = plsc.load_gather(table, [idx + LEN_PAD])
                    ve, vo = plsc.unpack(plsc.bitcast(gw1, jnp.bfloat16),
                                         format=plsc.PackFormat.INTERLEAVED)
                    a2 = a2 + ve * w
                    a3 = a3 + vo * w
            return (a0, a1, a2, a3)

        def pair_body(p, _):
            nm = wid * n_pairs + p
            pltpu.sync_copy(value_hbm.at[nm], table)

            def qc_body(qc, _):
                qb = pl.multiple_of(qc * QC, QC)
                pltpu.sync_copy(xyz_hbm.at[nm, :, pl.ds(qb, QC)], xyzv)
                pltpu.sync_copy(att_hbm.at[nm, :, pl.ds(qb, QC)], attv)

                def qv_body(qv, _):
                    qo = pl.multiple_of(qv * 16, 16)
                    z = jnp.zeros((16,), jnp.float32)
                    acc = lax.fori_loop(
                        0, LP, lambda lp, c: lp_body((lp, qo), c), (z, z, z, z))
                    for dd in range(DH):
                        outv[dd, pl.ds(qb + qo, 16)] = acc[dd]
                    return 0

                lax.fori_loop(0, QC // 16, qv_body, 0)
                return 0

            lax.fori_loop(0, LQ // QC, qc_body, 0)
            pltpu.sync_copy(outv, out_hbm.at[nm])
            return 0

        lax.fori_loop(0, n_pairs, pair_body, 0)

    return body(value_t, xyz, att, lvlc)


def _out_proj_body(s_ref, w_ref, b_ref, o_ref):
    y = lax.dot_general(s_ref[0], w_ref[...], (((0,), (0,)), ((), ())),
                        preferred_element_type=jnp.float32, precision=lax.Precision.HIGHEST)
    o_ref[...] = (y + b_ref[...])[None]


def _out_proj(sc_out, W, b):
    return pl.pallas_call(
        _out_proj_body,
        grid=(N_B,),
        in_specs=[
            pl.BlockSpec((1, D_MODEL, LQ), lambda n: (n, 0, 0)),
            pl.BlockSpec((D_MODEL, D_MODEL), lambda n: (0, 0)),
            pl.BlockSpec((D_MODEL,), lambda n: (0,)),
        ],
        out_specs=pl.BlockSpec((1, LQ, D_MODEL), lambda n: (n, 0, 0)),
        out_shape=jax.ShapeDtypeStruct((N_B, LQ, D_MODEL), jnp.float32),
    )(sc_out, W, b)


def _consts():
    sc = np.zeros((12, 3072), np.float32)
    for m in range(MH):
        for lvl in range(N_LEVELS):
            H, W = SPATIAL[lvl]
            s3 = (W, H, N_FRAMES)
            for p in range(N_POINTS):
                for c in range(3):
                    sc[lvl * 3 + c, ((m * N_LEVELS + lvl) * N_POINTS + p) * 3 + c] = s3[c]
    gsum = np.zeros((1024, MH), np.float32)
    gexp = np.zeros((MH, 1024), np.float32)
    for i in range(1024):
        gsum[i, i // 16] = 1.0
        gexp[i // 16, i] = 1.0
    return jnp.asarray(sc), jnp.asarray(gsum), jnp.asarray(gexp)


def kernel(query, reference_points, input_flatten, input_spatial_shapes,
           input_level_start_index, W_value, b_value, W_offsets, b_offsets,
           W_attn, b_attn, W_out, b_out):
    scale_mat, gsum, gexp = _consts()
    perm = np.concatenate([np.arange(0, D_MODEL, 2), np.arange(1, D_MODEL, 2)])
    value_t = _value_proj(input_flatten, W_value[:, perm], b_value[perm])
    rp12 = reference_points.reshape(N_B, LQ, 12)
    xyz, att = _payload(query, rp12, W_offsets, b_offsets - 0.5, W_attn, b_attn,
                        scale_mat, gsum, gexp)
    lvlc = np.zeros((N_LEVELS * N_POINTS * 5, 16), np.int32)
    for lvl in range(N_LEVELS):
        H, W = SPATIAL[lvl]
        for pt in range(N_POINTS):
            lp = lvl * N_POINTS + pt
            lvlc[5 * lp + 0] = W
            lvlc[5 * lp + 1] = W - 1
            lvlc[5 * lp + 2] = H - 1
            lvlc[5 * lp + 3] = LSI[lvl]
    sc_out = _sc_sample(value_t.reshape(NM, (DH // 2) * LEN_PAD),
                        xyz.reshape(NM, N_LEVELS * N_POINTS * 3, LQ),
                        att.reshape(NM, N_LEVELS * N_POINTS, LQ),
                        jnp.asarray(lvlc))
    return _out_proj(sc_out.reshape(N_B, D_MODEL, LQ), W_out, b_out)
